# baseline (speedup 1.0000x reference)
import jax
import jax.numpy as jnp
from jax.experimental import pallas as pl
from jax.experimental.pallas import tpu as pltpu

_N = 8192
_T = 1024
_G = _N // _T
_DEPTH = 4


def _body(o_ref, z_ref, e_ref, sems):
    i = pl.program_id(0)
    j = jax.lax.rem(i + 1 + pl.program_id(1), _G)
    step = i * _G + pl.program_id(1)

    @pl.when(step == 0)
    def _init_z():
        z_ref[...] = jnp.zeros((_T, _T), jnp.float32)

    @pl.when(step == 1)
    def _init_e():
        r = jax.lax.broadcasted_iota(jnp.int32, (_T, _T), 0)
        c = jax.lax.broadcasted_iota(jnp.int32, (_T, _T), 1)
        e_ref[...] = (r == c).astype(jnp.float32)

    slot = jax.lax.rem(step, _DEPTH)
    dst = o_ref.at[pl.ds(i * _T, _T), pl.ds(j * _T, _T)]

    @pl.when(step >= _DEPTH)
    def _drain_one():
        pltpu.make_async_copy(z_ref, dst, sems.at[slot]).wait()

    @pl.when(i == j)
    def _fire_eye():
        pltpu.make_async_copy(e_ref, dst, sems.at[slot]).start()

    @pl.when(i != j)
    def _fire_zero():
        pltpu.make_async_copy(z_ref, dst, sems.at[slot]).start()

    @pl.when(step == _G * _G - 1)
    def _drain_all():
        for k in range(_DEPTH):
            pltpu.make_async_copy(z_ref, dst, sems.at[k]).wait()


def kernel(x):
    seq_len = x.shape[1]
    return pl.pallas_call(
        _body,
        grid=(seq_len // _T, _N // _T),
        out_specs=pl.BlockSpec(memory_space=pl.ANY),
        out_shape=jax.ShapeDtypeStruct((seq_len, _N), x.dtype),
        scratch_shapes=[
            pltpu.VMEM((_T, _T), jnp.float32),
            pltpu.VMEM((_T, _T), jnp.float32),
            pltpu.SemaphoreType.DMA((_DEPTH,)),
        ],
    )()


# speedup vs baseline: 1.0448x; 1.0448x over previous
import jax
import jax.numpy as jnp
from jax.experimental import pallas as pl
from jax.experimental.pallas import tpu as pltpu

_N = 8192
_T = 512
_G = _N // _T
_DEPTH = 4


def _body(o_ref, z_ref, e_ref, sems):
    i = pl.program_id(0)
    j = jax.lax.rem(i + 1 + pl.program_id(1), _G)
    step = i * _G + pl.program_id(1)

    @pl.when(step == 0)
    def _init_z():
        z_ref[...] = jnp.zeros((_T, _T), jnp.float32)

    @pl.when(step == 1)
    def _init_e():
        r = jax.lax.broadcasted_iota(jnp.int32, (_T, _T), 0)
        c = jax.lax.broadcasted_iota(jnp.int32, (_T, _T), 1)
        e_ref[...] = (r == c).astype(jnp.float32)

    slot = jax.lax.rem(step, _DEPTH)
    dst = o_ref.at[pl.ds(i * _T, _T), pl.ds(j * _T, _T)]

    @pl.when(step >= _DEPTH)
    def _drain_one():
        pltpu.make_async_copy(z_ref, dst, sems.at[slot]).wait()

    @pl.when(i == j)
    def _fire_eye():
        pltpu.make_async_copy(e_ref, dst, sems.at[slot]).start()

    @pl.when(i != j)
    def _fire_zero():
        pltpu.make_async_copy(z_ref, dst, sems.at[slot]).start()

    @pl.when(step == _G * _G - 1)
    def _drain_all():
        for k in range(_DEPTH):
            pltpu.make_async_copy(z_ref, dst, sems.at[k]).wait()


def kernel(x):
    seq_len = x.shape[1]
    return pl.pallas_call(
        _body,
        grid=(seq_len // _T, _N // _T),
        out_specs=pl.BlockSpec(memory_space=pl.ANY),
        out_shape=jax.ShapeDtypeStruct((seq_len, _N), x.dtype),
        scratch_shapes=[
            pltpu.VMEM((_T, _T), jnp.float32),
            pltpu.VMEM((_T, _T), jnp.float32),
            pltpu.SemaphoreType.DMA((_DEPTH,)),
        ],
    )()
